# baseline (speedup 1.0000x reference)
import functools

import jax
import jax.numpy as jnp
from jax import lax
from jax.experimental import pallas as pl
from jax.experimental.pallas import tpu as pltpu
from jax.experimental.pallas import tpu_sc as plsc

NEG_SLOPE = 0.2
N = 10000
E = 320000
D = 128
HP = 16
GW = D + HP
NC, NS = 2, 16
NW = NC * NS
EPT = E // NW
CH = 20
NCHUNK = EPT // CH
NBUF = 4
ROWS_PER_TILE = 624
TAIL_START = ROWS_PER_TILE * NS
TAIL_ROWS = N - TAIL_START
BN = 2000


def _prep_body(x_ref, w_ref, avs_ref, avd_ref, sel_ref, g_ref, ad_ref):
    h = jnp.dot(x_ref[...], w_ref[...], preferred_element_type=jnp.float32)
    g_ref[:, :D] = h
    g_ref[:, D:] = jnp.dot(h * avs_ref[...], sel_ref[...],
                           preferred_element_type=jnp.float32)
    ad_ref[...] = jnp.dot(h * avd_ref[...], sel_ref[...],
                          preferred_element_type=jnp.float32)


def _tc_prep(x, w, av_s, av_d, sel):
    n = x.shape[0]
    return pl.pallas_call(
        _prep_body,
        grid=(n // BN,),
        in_specs=[
            pl.BlockSpec((BN, D), lambda i: (i, 0)),
            pl.BlockSpec((D, D), lambda i: (0, 0)),
            pl.BlockSpec((1, D), lambda i: (0, 0)),
            pl.BlockSpec((1, D), lambda i: (0, 0)),
            pl.BlockSpec((D, HP), lambda i: (0, 0)),
        ],
        out_specs=[
            pl.BlockSpec((BN, GW), lambda i: (i, 0)),
            pl.BlockSpec((BN, HP), lambda i: (i, 0)),
        ],
        out_shape=[
            jax.ShapeDtypeStruct((n, GW), jnp.float32),
            jax.ShapeDtypeStruct((n, HP), jnp.float32),
        ],
    )(x, w, av_s, av_d, sel)


def _combine_body(p_ref, g_ref, ad_ref, hm_ref, r_ref, b_ref, out_ref):
    al = g_ref[:, D:] + ad_ref[...]
    w_self = jnp.exp(jnp.maximum(al, NEG_SLOPE * al)) * hm_ref[...]
    acc = p_ref[0] + p_ref[1]
    msg = (acc[:, :D]
           + jnp.dot(w_self, r_ref[...], preferred_element_type=jnp.float32)
           * g_ref[:, :D])
    den = acc[:, D:] + w_self
    recip = 1.0 / (den + 1e-16)
    out_ref[...] = (msg * jnp.dot(recip, r_ref[...],
                                  preferred_element_type=jnp.float32)
                    + b_ref[...])


def _tc_combine(part, g, ad, hm, r, b):
    n = g.shape[0]
    return pl.pallas_call(
        _combine_body,
        grid=(n // BN,),
        in_specs=[
            pl.BlockSpec((NC, BN, GW), lambda i: (0, i, 0)),
            pl.BlockSpec((BN, GW), lambda i: (i, 0)),
            pl.BlockSpec((BN, HP), lambda i: (i, 0)),
            pl.BlockSpec((1, HP), lambda i: (0, 0)),
            pl.BlockSpec((HP, D), lambda i: (0, 0)),
            pl.BlockSpec((1, D), lambda i: (0, 0)),
        ],
        out_specs=pl.BlockSpec((BN, D), lambda i: (i, 0)),
        out_shape=jax.ShapeDtypeStruct((n, D), jnp.float32),
    )(part, g, ad, hm, r, b)


def _mid_body(p_ref, g_ref, ad_ref, hm_ref, r_ref, b_ref, w_ref, avs_ref,
              avd_ref, g2_ref, ad2_ref):
    al = g_ref[:, D:] + ad_ref[...]
    w_self = jnp.exp(jnp.maximum(al, NEG_SLOPE * al)) * hm_ref[...]
    acc = p_ref[0] + p_ref[1]
    msg = (acc[:, :D]
           + jnp.dot(w_self, r_ref[...], preferred_element_type=jnp.float32)
           * g_ref[:, :D])
    den = acc[:, D:] + w_self
    recip = 1.0 / (den + 1e-16)
    x = (msg * jnp.dot(recip, r_ref[...], preferred_element_type=jnp.float32)
         + b_ref[...])
    x = jnp.where(x > 0, x, jnp.exp(x) - 1.0)
    h = jnp.dot(x, w_ref[...], preferred_element_type=jnp.float32)
    g2_ref[:, :D] = h
    g2_ref[:, D:] = jnp.dot(h, avs_ref[...], preferred_element_type=jnp.float32)
    ad2_ref[...] = jnp.dot(h, avd_ref[...], preferred_element_type=jnp.float32)


def _tc_mid(part, g, ad, hm, r, b, w, a2s, a2d):
    n = g.shape[0]
    return pl.pallas_call(
        _mid_body,
        grid=(n // BN,),
        in_specs=[
            pl.BlockSpec((NC, BN, GW), lambda i: (0, i, 0)),
            pl.BlockSpec((BN, GW), lambda i: (i, 0)),
            pl.BlockSpec((BN, HP), lambda i: (i, 0)),
            pl.BlockSpec((1, HP), lambda i: (0, 0)),
            pl.BlockSpec((HP, D), lambda i: (0, 0)),
            pl.BlockSpec((1, D), lambda i: (0, 0)),
            pl.BlockSpec((D, D), lambda i: (0, 0)),
            pl.BlockSpec((D, HP), lambda i: (0, 0)),
            pl.BlockSpec((D, HP), lambda i: (0, 0)),
        ],
        out_specs=[
            pl.BlockSpec((BN, GW), lambda i: (i, 0)),
            pl.BlockSpec((BN, HP), lambda i: (i, 0)),
        ],
        out_shape=[
            jax.ShapeDtypeStruct((n, GW), jnp.float32),
            jax.ShapeDtypeStruct((n, HP), jnp.float32),
        ],
    )(part, g, ad, hm, r, b, w, a2s, a2d)


def _sc_edge_kernel(heads):
    mesh = plsc.VectorSubcoreMesh(core_axis_name="c", subcore_axis_name="s",
                                  num_cores=NC, num_subcores=NS)

    @functools.partial(
        pl.kernel,
        mesh=mesh,
        compiler_params=pltpu.CompilerParams(use_tc_tiling_on_sc=False),
        out_type=jax.ShapeDtypeStruct((NC, N, GW), jnp.float32),
        scratch_types=[
            pltpu.VMEM_SHARED((N, GW), jnp.float32),
            pltpu.VMEM((NCHUNK, CH), jnp.int32),
            pltpu.VMEM((NCHUNK, CH), jnp.int32),
            [pltpu.VMEM((CH, GW), jnp.float32)] * NBUF,
            [pltpu.VMEM((CH, HP), jnp.float32)] * NBUF,
            [pltpu.SemaphoreType.DMA] * NBUF,
            [pltpu.SemaphoreType.DMA] * NBUF,
        ],
    )
    def edge_kernel(g_hbm, ad_hbm, src_hbm, dst_hbm, z_hbm, out_hbm,
                    acc, sidx, didx, gbufs, dbufs, gsems, ssems):
        c = lax.axis_index("c")
        s = lax.axis_index("s")
        wid = s * NC + c
        pltpu.sync_copy(z_hbm.at[pl.ds(s * ROWS_PER_TILE, ROWS_PER_TILE)],
                        acc.at[pl.ds(s * ROWS_PER_TILE, ROWS_PER_TILE)])
        @pl.when(s == NS - 1)
        def _zero_tail():
            pltpu.sync_copy(z_hbm.at[pl.ds(TAIL_START, TAIL_ROWS)],
                            acc.at[pl.ds(TAIL_START, TAIL_ROWS)])
        pltpu.sync_copy(src_hbm.at[wid], sidx)
        pltpu.sync_copy(dst_hbm.at[wid], didx)
        plsc.subcore_barrier()

        lanes = lax.iota(jnp.int32, HP)
        bcast_dn = lax.GatherDimensionNumbers(
            offset_dims=(), collapsed_slice_dims=(0,), start_index_map=(0,))

        def issue_gather(b, k):
            pltpu.async_copy(g_hbm.at[sidx.at[k]], gbufs[b], gsems[b])
            pltpu.async_copy(ad_hbm.at[didx.at[k]], dbufs[b], gsems[b])

        def wait_gather(b, k):
            pltpu.make_async_copy(g_hbm.at[sidx.at[k]], gbufs[b],
                                  gsems[b]).wait()
            pltpu.make_async_copy(ad_hbm.at[didx.at[k]], dbufs[b],
                                  gsems[b]).wait()

        def issue_scatter(b, k):
            pltpu.async_copy(gbufs[b], acc.at[didx.at[k]], ssems[b], add=True)

        def wait_scatter(b, k):
            pltpu.make_async_copy(gbufs[b], acc.at[didx.at[k]],
                                  ssems[b]).wait()

        def compute(b):
            gbuf, dbuf = gbufs[b], dbufs[b]
            for e in range(CH):
                a = gbuf[e, pl.ds(D, HP)] + dbuf[e, :]
                a = jnp.maximum(a, NEG_SLOPE * a)
                w = jnp.where(lanes < heads, jnp.exp(a), 0.0)
                gbuf[e, pl.ds(D, HP)] = w
                for j in range(D // HP):
                    wj = lax.gather(
                        w, jnp.full((HP, 1), j % heads, jnp.int32), bcast_dn,
                        slice_sizes=(1,),
                        mode=lax.GatherScatterMode.PROMISE_IN_BOUNDS)
                    gbuf[e, pl.ds(HP * j, HP)] = gbuf[e, pl.ds(HP * j, HP)] * wj

        def process(b, k, prefetch):
            wait_gather(b, k)
            compute(b)
            issue_scatter(b, k)
            if prefetch:
                bp = (b + 2) % NBUF
                wait_scatter(bp, k - 2)
                issue_gather(bp, k + 2)

        issue_gather(0, 0)
        issue_gather(1, 1)
        process(0, 0, False)
        issue_gather(2, 2)
        process(1, 1, False)
        issue_gather(3, 3)
        process(2, 2, True)
        process(3, 3, True)

        def ring_iter(i, carry):
            for b in range(NBUF):
                process(b, i * NBUF + b, True)
            return carry

        _ = lax.fori_loop(1, NCHUNK // NBUF - 1, ring_iter, 0)
        for b in range(NBUF):
            process(b, NCHUNK - NBUF + b, b < 2)
        for b in range(NBUF):
            wait_scatter(b, NCHUNK - NBUF + b)
        plsc.subcore_barrier()
        pltpu.sync_copy(acc.at[pl.ds(s * ROWS_PER_TILE, ROWS_PER_TILE)],
                        out_hbm.at[c].at[pl.ds(s * ROWS_PER_TILE,
                                               ROWS_PER_TILE)])
        @pl.when(s == NS - 1)
        def _out_tail():
            pltpu.sync_copy(acc.at[pl.ds(TAIL_START, TAIL_ROWS)],
                            out_hbm.at[c].at[pl.ds(TAIL_START, TAIL_ROWS)])

    return edge_kernel


_sc_edge_8 = _sc_edge_kernel(8)
_sc_edge_1 = _sc_edge_kernel(1)


def kernel(x, edge_index, w1, att_src1, att_dst1, b1, w2, att_src2,
           att_dst2, b2):
    f32 = jnp.float32
    src = edge_index[0].reshape(NW, NCHUNK, CH)
    dst = edge_index[1].reshape(NW, NCHUNK, CH)

    ch_of = jnp.arange(D, dtype=jnp.int32) // HP
    hd = jnp.arange(HP, dtype=jnp.int32)
    sel = (ch_of[:, None] == hd[None, :]).astype(f32)
    r1 = sel.T
    r2 = (hd[:, None] == 0).astype(f32) * jnp.ones((1, D), f32)
    hm8 = (hd < 8).astype(f32).reshape(1, HP)
    hm1 = (hd < 1).astype(f32).reshape(1, HP)
    zeros_acc = jnp.zeros((N, GW), f32)

    av_s1 = att_src1.reshape(1, D)
    av_d1 = att_dst1.reshape(1, D)
    a2s = att_src2.reshape(D, 1) * (hd[None, :] == 0).astype(f32)
    a2d = att_dst2.reshape(D, 1) * (hd[None, :] == 0).astype(f32)

    g1, ad1 = _tc_prep(x, w1, av_s1, av_d1, sel)
    part1 = _sc_edge_8(g1, ad1, src, dst, zeros_acc)
    g2, ad2 = _tc_mid(part1, g1, ad1, hm8, r1, b1.reshape(1, D),
                      w2, a2s, a2d)
    part2 = _sc_edge_1(g2, ad2, src, dst, zeros_acc)
    out = _tc_combine(part2, g2, ad2, hm1, r2, b2.reshape(1, D))
    return out


# speedup vs baseline: 1.1346x; 1.1346x over previous
import functools

import jax
import jax.numpy as jnp
from jax import lax
from jax.experimental import pallas as pl
from jax.experimental.pallas import tpu as pltpu
from jax.experimental.pallas import tpu_sc as plsc

NEG_SLOPE = 0.2
N = 10000
E = 320000
D = 128
HP = 16
GW = D + HP
NC, NS = 2, 16
NW = NC * NS
EPT = E // NW
CH = 50
NCHUNK = EPT // CH
NBUF = 2
ZROWS = 48
ROWS_PER_TILE = 624
TAIL_START = ROWS_PER_TILE * NS
TAIL_ROWS = N - TAIL_START
BN = 2000


def _prep_body(x_ref, w_ref, avs_ref, avd_ref, sel_ref, g_ref, ad_ref):
    h = jnp.dot(x_ref[...], w_ref[...], preferred_element_type=jnp.float32)
    g_ref[:, :D] = h
    g_ref[:, D:] = jnp.dot(h * avs_ref[...], sel_ref[...],
                           preferred_element_type=jnp.float32)
    ad_ref[...] = jnp.dot(h * avd_ref[...], sel_ref[...],
                          preferred_element_type=jnp.float32)


def _tc_prep(x, w, av_s, av_d, sel):
    n = x.shape[0]
    return pl.pallas_call(
        _prep_body,
        grid=(n // BN,),
        in_specs=[
            pl.BlockSpec((BN, D), lambda i: (i, 0)),
            pl.BlockSpec((D, D), lambda i: (0, 0)),
            pl.BlockSpec((1, D), lambda i: (0, 0)),
            pl.BlockSpec((1, D), lambda i: (0, 0)),
            pl.BlockSpec((D, HP), lambda i: (0, 0)),
        ],
        out_specs=[
            pl.BlockSpec((BN, GW), lambda i: (i, 0)),
            pl.BlockSpec((BN, HP), lambda i: (i, 0)),
        ],
        out_shape=[
            jax.ShapeDtypeStruct((n, GW), jnp.float32),
            jax.ShapeDtypeStruct((n, HP), jnp.float32),
        ],
    )(x, w, av_s, av_d, sel)


def _combine_body(p_ref, g_ref, ad_ref, hm_ref, r_ref, b_ref, out_ref):
    al = g_ref[:, D:] + ad_ref[...]
    w_self = jnp.exp(jnp.maximum(al, NEG_SLOPE * al)) * hm_ref[...]
    acc = p_ref[0] + p_ref[1]
    msg = (acc[:, :D]
           + jnp.dot(w_self, r_ref[...], preferred_element_type=jnp.float32)
           * g_ref[:, :D])
    den = acc[:, D:] + w_self
    recip = 1.0 / (den + 1e-16)
    out_ref[...] = (msg * jnp.dot(recip, r_ref[...],
                                  preferred_element_type=jnp.float32)
                    + b_ref[...])


def _tc_combine(part, g, ad, hm, r, b):
    n = g.shape[0]
    return pl.pallas_call(
        _combine_body,
        grid=(n // BN,),
        in_specs=[
            pl.BlockSpec((NC, BN, GW), lambda i: (0, i, 0)),
            pl.BlockSpec((BN, GW), lambda i: (i, 0)),
            pl.BlockSpec((BN, HP), lambda i: (i, 0)),
            pl.BlockSpec((1, HP), lambda i: (0, 0)),
            pl.BlockSpec((HP, D), lambda i: (0, 0)),
            pl.BlockSpec((1, D), lambda i: (0, 0)),
        ],
        out_specs=pl.BlockSpec((BN, D), lambda i: (i, 0)),
        out_shape=jax.ShapeDtypeStruct((n, D), jnp.float32),
    )(part, g, ad, hm, r, b)


def _mid_body(p_ref, g_ref, ad_ref, hm_ref, r_ref, b_ref, w_ref, avs_ref,
              avd_ref, g2_ref, ad2_ref):
    al = g_ref[:, D:] + ad_ref[...]
    w_self = jnp.exp(jnp.maximum(al, NEG_SLOPE * al)) * hm_ref[...]
    acc = p_ref[0] + p_ref[1]
    msg = (acc[:, :D]
           + jnp.dot(w_self, r_ref[...], preferred_element_type=jnp.float32)
           * g_ref[:, :D])
    den = acc[:, D:] + w_self
    recip = 1.0 / (den + 1e-16)
    x = (msg * jnp.dot(recip, r_ref[...], preferred_element_type=jnp.float32)
         + b_ref[...])
    x = jnp.where(x > 0, x, jnp.exp(x) - 1.0)
    h = jnp.dot(x, w_ref[...], preferred_element_type=jnp.float32)
    g2_ref[:, :D] = h
    g2_ref[:, D:] = jnp.dot(h, avs_ref[...], preferred_element_type=jnp.float32)
    ad2_ref[...] = jnp.dot(h, avd_ref[...], preferred_element_type=jnp.float32)


def _tc_mid(part, g, ad, hm, r, b, w, a2s, a2d):
    n = g.shape[0]
    return pl.pallas_call(
        _mid_body,
        grid=(n // BN,),
        in_specs=[
            pl.BlockSpec((NC, BN, GW), lambda i: (0, i, 0)),
            pl.BlockSpec((BN, GW), lambda i: (i, 0)),
            pl.BlockSpec((BN, HP), lambda i: (i, 0)),
            pl.BlockSpec((1, HP), lambda i: (0, 0)),
            pl.BlockSpec((HP, D), lambda i: (0, 0)),
            pl.BlockSpec((1, D), lambda i: (0, 0)),
            pl.BlockSpec((D, D), lambda i: (0, 0)),
            pl.BlockSpec((D, HP), lambda i: (0, 0)),
            pl.BlockSpec((D, HP), lambda i: (0, 0)),
        ],
        out_specs=[
            pl.BlockSpec((BN, GW), lambda i: (i, 0)),
            pl.BlockSpec((BN, HP), lambda i: (i, 0)),
        ],
        out_shape=[
            jax.ShapeDtypeStruct((n, GW), jnp.float32),
            jax.ShapeDtypeStruct((n, HP), jnp.float32),
        ],
    )(part, g, ad, hm, r, b, w, a2s, a2d)


def _sc_edge_kernel(heads):
    mesh = plsc.VectorSubcoreMesh(core_axis_name="c", subcore_axis_name="s",
                                  num_cores=NC, num_subcores=NS)

    @functools.partial(
        pl.kernel,
        mesh=mesh,
        compiler_params=pltpu.CompilerParams(use_tc_tiling_on_sc=False),
        out_type=jax.ShapeDtypeStruct((NC, N, GW), jnp.float32),
        scratch_types=[
            pltpu.VMEM_SHARED((N, GW), jnp.float32),
            pltpu.VMEM((NCHUNK, CH), jnp.int32),
            pltpu.VMEM((NCHUNK, CH), jnp.int32),
            [pltpu.VMEM((CH, GW), jnp.float32)] * NBUF,
            [pltpu.VMEM((CH, HP), jnp.float32)] * NBUF,
            [pltpu.SemaphoreType.DMA] * NBUF,
            [pltpu.SemaphoreType.DMA] * NBUF,
        ],
    )
    def edge_kernel(g_hbm, ad_hbm, src_hbm, dst_hbm, out_hbm,
                    acc, sidx, didx, gbufs, dbufs, gsems, ssems):
        c = lax.axis_index("c")
        s = lax.axis_index("s")
        wid = s * NC + c
        zero = jnp.zeros((HP,), jnp.float32)
        for r in range(ZROWS):
            for v in range(GW // HP):
                gbufs[0][r, pl.ds(HP * v, HP)] = zero
        for blk in range(ROWS_PER_TILE // ZROWS):
            pltpu.sync_copy(
                gbufs[0].at[pl.ds(0, ZROWS)],
                acc.at[pl.ds(s * ROWS_PER_TILE + blk * ZROWS, ZROWS)])
        @pl.when(s == NS - 1)
        def _zero_tail():
            pltpu.sync_copy(gbufs[0].at[pl.ds(0, TAIL_ROWS)],
                            acc.at[pl.ds(TAIL_START, TAIL_ROWS)])
        pltpu.sync_copy(src_hbm.at[wid], sidx)
        pltpu.sync_copy(dst_hbm.at[wid], didx)

        lanes = lax.iota(jnp.int32, HP)
        bcast_dn = lax.GatherDimensionNumbers(
            offset_dims=(), collapsed_slice_dims=(0,), start_index_map=(0,))

        def issue_gather(b, k):
            pltpu.async_copy(g_hbm.at[sidx.at[k]], gbufs[b], gsems[b])
            pltpu.async_copy(ad_hbm.at[didx.at[k]], dbufs[b], gsems[b])

        def wait_gather(b, k):
            pltpu.make_async_copy(g_hbm.at[sidx.at[k]], gbufs[b],
                                  gsems[b]).wait()
            pltpu.make_async_copy(ad_hbm.at[didx.at[k]], dbufs[b],
                                  gsems[b]).wait()

        def issue_scatter(b, k):
            pltpu.async_copy(gbufs[b], acc.at[didx.at[k]], ssems[b], add=True)

        def wait_scatter(b, k):
            pltpu.make_async_copy(gbufs[b], acc.at[didx.at[k]],
                                  ssems[b]).wait()

        def compute(b):
            gbuf, dbuf = gbufs[b], dbufs[b]
            for e in range(CH):
                a = gbuf[e, pl.ds(D, HP)] + dbuf[e, :]
                a = jnp.maximum(a, NEG_SLOPE * a)
                w = jnp.where(lanes < heads, jnp.exp(a), 0.0)
                gbuf[e, pl.ds(D, HP)] = w
                for j in range(D // HP):
                    wj = lax.gather(
                        w, jnp.full((HP, 1), j % heads, jnp.int32), bcast_dn,
                        slice_sizes=(1,),
                        mode=lax.GatherScatterMode.PROMISE_IN_BOUNDS)
                    gbuf[e, pl.ds(HP * j, HP)] = gbuf[e, pl.ds(HP * j, HP)] * wj

        issue_gather(0, 0)
        issue_gather(1, 1)
        plsc.subcore_barrier()

        def process(b, k):
            bp = (b + 1) % NBUF
            wait_gather(b, k)
            wait_scatter(bp, k - 1)
            issue_gather(bp, k + 1)
            compute(b)
            issue_scatter(b, k)

        wait_gather(0, 0)
        compute(0)
        issue_scatter(0, 0)

        def ring_iter(i, carry):
            process(1, 2 * i + 1)
            process(0, 2 * i + 2)
            return carry

        _ = lax.fori_loop(0, (NCHUNK - 2) // 2, ring_iter, 0)
        wait_gather(1, NCHUNK - 1)
        compute(1)
        issue_scatter(1, NCHUNK - 1)
        wait_scatter(0, NCHUNK - 2)
        wait_scatter(1, NCHUNK - 1)
        plsc.subcore_barrier()
        pltpu.sync_copy(acc.at[pl.ds(s * ROWS_PER_TILE, ROWS_PER_TILE)],
                        out_hbm.at[c].at[pl.ds(s * ROWS_PER_TILE,
                                               ROWS_PER_TILE)])
        @pl.when(s == NS - 1)
        def _out_tail():
            pltpu.sync_copy(acc.at[pl.ds(TAIL_START, TAIL_ROWS)],
                            out_hbm.at[c].at[pl.ds(TAIL_START, TAIL_ROWS)])

    return edge_kernel


_sc_edge_8 = _sc_edge_kernel(8)
_sc_edge_1 = _sc_edge_kernel(1)


def kernel(x, edge_index, w1, att_src1, att_dst1, b1, w2, att_src2,
           att_dst2, b2):
    f32 = jnp.float32
    src = edge_index[0].reshape(NW, NCHUNK, CH)
    dst = edge_index[1].reshape(NW, NCHUNK, CH)

    ch_of = jnp.arange(D, dtype=jnp.int32) // HP
    hd = jnp.arange(HP, dtype=jnp.int32)
    sel = (ch_of[:, None] == hd[None, :]).astype(f32)
    r1 = sel.T
    r2 = (hd[:, None] == 0).astype(f32) * jnp.ones((1, D), f32)
    hm8 = (hd < 8).astype(f32).reshape(1, HP)
    hm1 = (hd < 1).astype(f32).reshape(1, HP)

    av_s1 = att_src1.reshape(1, D)
    av_d1 = att_dst1.reshape(1, D)
    a2s = att_src2.reshape(D, 1) * (hd[None, :] == 0).astype(f32)
    a2d = att_dst2.reshape(D, 1) * (hd[None, :] == 0).astype(f32)

    g1, ad1 = _tc_prep(x, w1, av_s1, av_d1, sel)
    part1 = _sc_edge_8(g1, ad1, src, dst)
    g2, ad2 = _tc_mid(part1, g1, ad1, hm8, r1, b1.reshape(1, D),
                      w2, a2s, a2d)
    part2 = _sc_edge_1(g2, ad2, src, dst)
    out = _tc_combine(part2, g2, ad2, hm1, r2, b2.reshape(1, D))
    return out
